# baseline (speedup 1.0000x reference)
import jax
import jax.numpy as jnp
from jax.experimental import pallas as pl
from jax.experimental.pallas import tpu as pltpu

_CH = 6400
_NBUF = 8


def _body(x1_hbm, x2_hbm, x3_hbm, w1_ref, w2_ref, w3_ref, b_ref, out_hbm,
          b1, b2, b3, bo, in_sem, out_sem):
    e = x1_hbm.shape[1]
    nchunks = e // _CH

    def in_copies(i, slot):
        c = i * _CH
        return (
            pltpu.make_async_copy(x1_hbm.at[:, pl.ds(c, _CH)], b1.at[slot],
                                  in_sem.at[slot, 0]),
            pltpu.make_async_copy(x2_hbm.at[:, pl.ds(c, _CH)], b2.at[slot],
                                  in_sem.at[slot, 1]),
            pltpu.make_async_copy(x3_hbm.at[:, pl.ds(c, _CH)], b3.at[slot],
                                  in_sem.at[slot, 2]),
        )

    def out_copy(i, slot):
        return pltpu.make_async_copy(bo.at[slot],
                                     out_hbm.at[:, pl.ds(i * _CH, _CH)],
                                     out_sem.at[slot])

    for s in range(_NBUF):
        for cp in in_copies(s, s):
            cp.start()

    dn = (((0,), (0,)), ((), ()))

    def step(i, carry):
        slot = jax.lax.rem(i, _NBUF)
        for cp in in_copies(i, slot):
            cp.wait()

        @pl.when(i >= _NBUF)
        def _():
            out_copy(i - _NBUF, slot).wait()

        acc = jax.lax.dot_general(w1_ref[...], b1[slot], dn,
                                  preferred_element_type=jnp.float32)
        acc = acc + jax.lax.dot_general(w2_ref[...], b2[slot], dn,
                                        preferred_element_type=jnp.float32)
        acc = acc + jax.lax.dot_general(w3_ref[...], b3[slot], dn,
                                        preferred_element_type=jnp.float32)
        acc = acc + b_ref[...][:, 0:1]
        bo[slot] = jnp.maximum(acc, 0.0)
        out_copy(i, slot).start()

        @pl.when(i + _NBUF < nchunks)
        def _():
            for cp in in_copies(i + _NBUF, slot):
                cp.start()

        return carry

    jax.lax.fori_loop(0, nchunks, step, 0)

    for s in range(_NBUF):
        i = nchunks - _NBUF + s
        out_copy(i, jax.lax.rem(i, _NBUF)).wait()


def kernel(f_src, f, sum_msg, w1, w2, w3, b):
    e, d_ndata = f_src.shape
    d_edata = f.shape[1]
    d_msg = sum_msg.shape[1]

    x1 = f_src.T
    x2 = f.T
    x3 = sum_msg.T
    bt = jnp.tile(b.reshape(d_msg, 1), (1, 128))

    out = pl.pallas_call(
        _body,
        in_specs=[
            pl.BlockSpec(memory_space=pl.ANY),
            pl.BlockSpec(memory_space=pl.ANY),
            pl.BlockSpec(memory_space=pl.ANY),
            pl.BlockSpec(memory_space=pltpu.MemorySpace.VMEM),
            pl.BlockSpec(memory_space=pltpu.MemorySpace.VMEM),
            pl.BlockSpec(memory_space=pltpu.MemorySpace.VMEM),
            pl.BlockSpec(memory_space=pltpu.MemorySpace.VMEM),
        ],
        out_specs=pl.BlockSpec(memory_space=pl.ANY),
        out_shape=jax.ShapeDtypeStruct((d_msg, e), jnp.float32),
        scratch_shapes=[
            pltpu.VMEM((_NBUF, d_ndata, _CH), jnp.float32),
            pltpu.VMEM((_NBUF, d_edata, _CH), jnp.float32),
            pltpu.VMEM((_NBUF, d_msg, _CH), jnp.float32),
            pltpu.VMEM((_NBUF, d_msg, _CH), jnp.float32),
            pltpu.SemaphoreType.DMA((_NBUF, 3)),
            pltpu.SemaphoreType.DMA((_NBUF,)),
        ],
    )(x1, x2, x3, w1, w2, w3, bt)
    return out.T


# speedup vs baseline: 1.0007x; 1.0007x over previous
import jax
import jax.numpy as jnp
from jax.experimental import pallas as pl
from jax.experimental.pallas import tpu as pltpu

_CH = 12800
_NBUF = 4


def _body(x1_hbm, x2_hbm, x3_hbm, w1_ref, w2_ref, w3_ref, b_ref, out_hbm,
          b1, b2, b3, bo, in_sem, out_sem):
    e = x1_hbm.shape[1]
    nchunks = e // _CH

    def in_copies(i, slot):
        c = i * _CH
        return (
            pltpu.make_async_copy(x1_hbm.at[:, pl.ds(c, _CH)], b1.at[slot],
                                  in_sem.at[slot, 0]),
            pltpu.make_async_copy(x2_hbm.at[:, pl.ds(c, _CH)], b2.at[slot],
                                  in_sem.at[slot, 1]),
            pltpu.make_async_copy(x3_hbm.at[:, pl.ds(c, _CH)], b3.at[slot],
                                  in_sem.at[slot, 2]),
        )

    def out_copy(i, slot):
        return pltpu.make_async_copy(bo.at[slot],
                                     out_hbm.at[:, pl.ds(i * _CH, _CH)],
                                     out_sem.at[slot])

    for s in range(_NBUF):
        for cp in in_copies(s, s):
            cp.start()

    dn = (((0,), (0,)), ((), ()))

    def step(i, carry):
        slot = jax.lax.rem(i, _NBUF)
        for cp in in_copies(i, slot):
            cp.wait()

        @pl.when(i >= _NBUF)
        def _():
            out_copy(i - _NBUF, slot).wait()

        acc = jax.lax.dot_general(w1_ref[...], b1[slot], dn,
                                  preferred_element_type=jnp.float32)
        acc = acc + jax.lax.dot_general(w2_ref[...], b2[slot], dn,
                                        preferred_element_type=jnp.float32)
        acc = acc + jax.lax.dot_general(w3_ref[...], b3[slot], dn,
                                        preferred_element_type=jnp.float32)
        acc = acc + b_ref[...][:, 0:1]
        bo[slot] = jnp.maximum(acc, 0.0)
        out_copy(i, slot).start()

        @pl.when(i + _NBUF < nchunks)
        def _():
            for cp in in_copies(i + _NBUF, slot):
                cp.start()

        return carry

    jax.lax.fori_loop(0, nchunks, step, 0)

    for s in range(_NBUF):
        i = nchunks - _NBUF + s
        out_copy(i, jax.lax.rem(i, _NBUF)).wait()


def kernel(f_src, f, sum_msg, w1, w2, w3, b):
    e, d_ndata = f_src.shape
    d_edata = f.shape[1]
    d_msg = sum_msg.shape[1]

    x1 = f_src.T
    x2 = f.T
    x3 = sum_msg.T
    bt = jnp.tile(b.reshape(d_msg, 1), (1, 128))

    out = pl.pallas_call(
        _body,
        in_specs=[
            pl.BlockSpec(memory_space=pl.ANY),
            pl.BlockSpec(memory_space=pl.ANY),
            pl.BlockSpec(memory_space=pl.ANY),
            pl.BlockSpec(memory_space=pltpu.MemorySpace.VMEM),
            pl.BlockSpec(memory_space=pltpu.MemorySpace.VMEM),
            pl.BlockSpec(memory_space=pltpu.MemorySpace.VMEM),
            pl.BlockSpec(memory_space=pltpu.MemorySpace.VMEM),
        ],
        out_specs=pl.BlockSpec(memory_space=pl.ANY),
        out_shape=jax.ShapeDtypeStruct((d_msg, e), jnp.float32),
        scratch_shapes=[
            pltpu.VMEM((_NBUF, d_ndata, _CH), jnp.float32),
            pltpu.VMEM((_NBUF, d_edata, _CH), jnp.float32),
            pltpu.VMEM((_NBUF, d_msg, _CH), jnp.float32),
            pltpu.VMEM((_NBUF, d_msg, _CH), jnp.float32),
            pltpu.SemaphoreType.DMA((_NBUF, 3)),
            pltpu.SemaphoreType.DMA((_NBUF,)),
        ],
    )(x1, x2, x3, w1, w2, w3, bt)
    return out.T
